# baseline (speedup 1.0000x reference)
import functools

import jax
import jax.numpy as jnp
from jax import lax
from jax.experimental import pallas as pl
from jax.experimental.pallas import tpu as pltpu
from jax.experimental.pallas import tpu_sc as plsc

N_NODES = 10000
N_EDGES = 320000
H = 128
G = 128
EBLK = 16384
S0P = 10112
NW = 32
CH = N_EDGES // NW
L = 16
NEG = -3.4e38

_SC_PARAMS = pltpu.CompilerParams(needs_layout_passes=False)


def _sc_mesh():
    return plsc.VectorSubcoreMesh(
        core_axis_name="c", subcore_axis_name="s", num_cores=2, num_subcores=16)


def _a_body(fts_ref, w_ref, b_ref, ei_ref, l0_ref, l1_ref, idx_ref):
    lg = lax.dot_general(w_ref[...], fts_ref[...],
                         (((1,), (1,)), ((), ())))
    lg = lg + b_ref[...]
    l0_ref[...] = lg[0]
    l1_ref[...] = lg[1]
    idx_ref[...] = ei_ref[0]


def _edge_logits(edge_fts, W_edge, b_edge, edge_index):
    return pl.pallas_call(
        _a_body,
        grid=((N_EDGES + EBLK - 1) // EBLK,),
        in_specs=[
            pl.BlockSpec((EBLK, H), lambda i: (i, 0)),
            pl.BlockSpec((2, H), lambda i: (0, 0)),
            pl.BlockSpec((2, 1), lambda i: (0, 0)),
            pl.BlockSpec((2, EBLK), lambda i: (0, i)),
        ],
        out_specs=[
            pl.BlockSpec((EBLK,), lambda i: (i,)),
            pl.BlockSpec((EBLK,), lambda i: (i,)),
            pl.BlockSpec((EBLK,), lambda i: (i,)),
        ],
        out_shape=[
            jax.ShapeDtypeStruct((N_EDGES,), jnp.float32),
            jax.ShapeDtypeStruct((N_EDGES,), jnp.float32),
            jax.ShapeDtypeStruct((N_EDGES,), jnp.int32),
        ],
    )(edge_fts, W_edge, b_edge.reshape(2, 1), edge_index)


def _scatter_max16(acc, idx, val):
    def cond(act):
        return jnp.any(act)

    def body(act):
        cur = plsc.load_gather(acc, [idx])
        need = jnp.logical_and(act, val > cur)
        plsc.store_scatter(acc, [idx], val, mask=need)
        cur2 = plsc.load_gather(acc, [idx])
        return jnp.logical_and(need, val > cur2)

    act0 = val > plsc.load_gather(acc, [idx])
    lax.while_loop(cond, body, act0)


def _scatter_max16_pair(acc_a, idx_a, val_a, acc_b, idx_b, val_b):
    def cond(st):
        aa, ab = st
        return jnp.any(jnp.logical_or(aa, ab))

    def body(st):
        aa, ab = st
        cura = plsc.load_gather(acc_a, [idx_a])
        needa = jnp.logical_and(aa, val_a > cura)
        plsc.store_scatter(acc_a, [idx_a], val_a, mask=needa)
        curb = plsc.load_gather(acc_b, [idx_b])
        needb = jnp.logical_and(ab, val_b > curb)
        plsc.store_scatter(acc_b, [idx_b], val_b, mask=needb)
        cura2 = plsc.load_gather(acc_a, [idx_a])
        curb2 = plsc.load_gather(acc_b, [idx_b])
        return (jnp.logical_and(needa, val_a > cura2),
                jnp.logical_and(needb, val_b > curb2))

    aa0 = val_a > plsc.load_gather(acc_a, [idx_a])
    ab0 = val_b > plsc.load_gather(acc_b, [idx_b])
    lax.while_loop(cond, body, (aa0, ab0))


def _vfill(ref, n, value, dtype):
    def body(i, _):
        ref[pl.ds(i * L, L)] = jnp.full((L,), value, dtype)
        return 0

    lax.fori_loop(0, n // L, body, 0)


def _k1_partials(e_idx, l0, l1):
    @functools.partial(
        pl.kernel,
        out_type=(
            jax.ShapeDtypeStruct((NW, S0P), jnp.float32),
            jax.ShapeDtypeStruct((NW, S0P), jnp.float32),
        ),
        mesh=_sc_mesh(),
        compiler_params=_SC_PARAMS,
        scratch_types=[
            pltpu.VMEM((CH,), jnp.int32),
            pltpu.VMEM((CH,), jnp.float32),
            pltpu.VMEM((CH,), jnp.float32),
            pltpu.VMEM((S0P,), jnp.float32),
            pltpu.VMEM((S0P,), jnp.float32),
        ],
    )
    def k(idx_h, l0_h, l1_h, m0p_h, m1p_h,
          idx_v, l0_v, l1_v, m0a, m1a):
        wid = lax.axis_index("s") * 2 + lax.axis_index("c")
        base = wid * CH
        pltpu.sync_copy(idx_h.at[pl.ds(base, CH)], idx_v)
        pltpu.sync_copy(l0_h.at[pl.ds(base, CH)], l0_v)
        pltpu.sync_copy(l1_h.at[pl.ds(base, CH)], l1_v)
        _vfill(m0a, S0P, NEG, jnp.float32)
        _vfill(m1a, S0P, NEG, jnp.float32)

        def step(j, _):
            sl = pl.ds(j * L, L)
            idx = idx_v[sl]
            _scatter_max16_pair(m0a, idx, l0_v[sl], m1a, idx, l1_v[sl])
            return 0

        lax.fori_loop(0, CH // L, step, 0)
        pltpu.sync_copy(m0a, m0p_h.at[wid])
        pltpu.sync_copy(m1a, m1p_h.at[wid])

    return k(e_idx, l0, l1)


_CB = 2500


def _c1a_chunk(c, nf_ref, w_ref, b_ref, g0_ref, g1_ref, g2_ref, bv_ref):
    sl = pl.ds(c * _CB, _CB)
    x = nf_ref[sl, :]
    logits = lax.dot_general(x, w_ref[...],
                             (((1,), (1,)), ((), ())))
    logits = logits + b_ref[...]
    gt = jnp.concatenate([g0_ref[sl, :], g1_ref[sl, :], g2_ref[sl, :]],
                         axis=1)
    bv = bv_ref[sl, :]
    onehot = bv == lax.broadcasted_iota(jnp.int32, (_CB, G), 1)
    return logits, gt, onehot


def _c1a_body(tf_ref, nf_ref, w_ref, b_ref, g0_ref, g1_ref, g2_ref, bv_ref,
              states_ref, np_ref):
    def ph1(c, carry):
        m_n, gseg, dotn, bce1, bce2, n0c = carry
        logits, gt, onehot = _c1a_chunk(c, nf_ref, w_ref, b_ref, g0_ref,
                                        g1_ref, g2_ref, bv_ref)
        l0 = logits[:, 0:1]
        g0 = gt[:, 0:1]
        m_n = jnp.maximum(m_n, jnp.max(jnp.where(onehot, l0, NEG), axis=0,
                                       keepdims=True))
        gseg = gseg + jnp.sum(jnp.where(onehot, g0, 0.0), axis=0,
                              keepdims=True)
        dotn = dotn + jnp.sum(g0 * l0)
        l1 = logits[:, 1:2]
        g1 = gt[:, 1:2]
        bce1 = bce1 + jnp.sum(jnp.maximum(l1, 0.0) - l1 * g1 +
                              jnp.log1p(jnp.exp(-jnp.abs(l1))))
        l2 = logits[:, 2:3]
        g2 = gt[:, 2:3]
        bce2 = bce2 + jnp.sum(jnp.maximum(l2, 0.0) - l2 * g2 +
                              jnp.log1p(jnp.exp(-jnp.abs(l2))))
        n0c = n0c + jnp.sum(jnp.where(onehot[:, 0:1], 1.0, 0.0))
        return m_n, gseg, dotn, bce1, bce2, n0c

    init = (jnp.full((1, G), NEG, jnp.float32),
            jnp.zeros((1, G), jnp.float32),
            jnp.float32(0.0), jnp.float32(0.0), jnp.float32(0.0),
            jnp.float32(0.0))
    m_n, gseg, dotn, bce1, bce2, n0c = lax.fori_loop(
        0, N_NODES // _CB, ph1, init)

    def ph2(c, denom):
        logits, gt, onehot = _c1a_chunk(c, nf_ref, w_ref, b_ref, g0_ref,
                                        g1_ref, g2_ref, bv_ref)
        l0 = logits[:, 0:1]
        m_gath = jnp.sum(jnp.where(onehot, m_n, 0.0), axis=1, keepdims=True)
        denom = denom + jnp.sum(jnp.where(onehot, jnp.exp(l0 - m_gath), 0.0),
                                axis=0, keepdims=True)
        preds = jnp.concatenate(
            [(l0 >= m_gath).astype(jnp.float32),
             (logits[:, 1:2] > 0.0).astype(jnp.float32),
             (logits[:, 2:3] > 0.0).astype(jnp.float32)], axis=1)
        states_ref[pl.ds(c * _CB, _CB), :] = jnp.where(tf_ref[0] != 0, gt,
                                                       preds)
        return denom

    denom = lax.fori_loop(0, N_NODES // _CB, ph2,
                          jnp.zeros((1, G), jnp.float32))

    loss_n0 = (-dotn + jnp.sum(gseg * m_n) +
               jnp.sum(gseg * jnp.log(denom + 1e-20))) / G
    loss_node = loss_n0 + bce1 / N_NODES + bce2 / N_NODES
    ii = lax.broadcasted_iota(jnp.int32, (1, G), 1)
    np_ref[...] = jnp.where(ii == 0, loss_node,
                            jnp.where(ii == 1, n0c, 0.0))


def _c1a(node_fts, W_node, b_node, gtn0, gtn1, gtn2, batch_vec, tf_i):
    return pl.pallas_call(
        _c1a_body,
        grid=(1,),
        in_specs=[
            pl.BlockSpec(memory_space=pltpu.SMEM),
            pl.BlockSpec((N_NODES, H), lambda i: (0, 0)),
            pl.BlockSpec((3, H), lambda i: (0, 0)),
            pl.BlockSpec((1, 3), lambda i: (0, 0)),
            pl.BlockSpec((N_NODES, 1), lambda i: (0, 0)),
            pl.BlockSpec((N_NODES, 1), lambda i: (0, 0)),
            pl.BlockSpec((N_NODES, 1), lambda i: (0, 0)),
            pl.BlockSpec((N_NODES, 1), lambda i: (0, 0)),
        ],
        out_specs=[
            pl.BlockSpec((N_NODES, 3), lambda i: (0, 0)),
            pl.BlockSpec((1, G), lambda i: (0, 0)),
        ],
        out_shape=[
            jax.ShapeDtypeStruct((N_NODES, 3), jnp.float32),
            jax.ShapeDtypeStruct((1, G), jnp.float32),
        ],
    )(tf_i, node_fts, W_node, b_node.reshape(1, 3), gtn0, gtn1, gtn2,
      batch_vec.reshape(N_NODES, 1))


def _c1b_body(m0p_ref, m1p_ref, bv_ref, m0f_ref, m2f_ref):
    m0f_ref[...] = jnp.max(m0p_ref[...], axis=0, keepdims=True)
    m1f = jnp.max(m1p_ref[...], axis=0, keepdims=True)
    gmask = bv_ref[...] == lax.broadcasted_iota(jnp.int32, (G, S0P), 0)
    m2f_ref[...] = jnp.max(jnp.where(gmask, m1f, NEG), axis=1, keepdims=True)


def _c1b(m0p, m1p, bv_row):
    return pl.pallas_call(
        _c1b_body,
        grid=(1,),
        in_specs=[
            pl.BlockSpec((NW, S0P), lambda i: (0, 0)),
            pl.BlockSpec((NW, S0P), lambda i: (0, 0)),
            pl.BlockSpec((1, S0P), lambda i: (0, 0)),
        ],
        out_specs=[
            pl.BlockSpec((1, S0P), lambda i: (0, 0)),
            pl.BlockSpec((G, 1), lambda i: (0, 0)),
        ],
        out_shape=[
            jax.ShapeDtypeStruct((1, S0P), jnp.float32),
            jax.ShapeDtypeStruct((G, 1), jnp.float32),
        ],
    )(m0p, m1p, bv_row)


def _k3_denoms_states(e_idx, l0, l1, g0, g1, batch_vec, m0f, m2f, tf16):
    @functools.partial(
        pl.kernel,
        out_type=(
            jax.ShapeDtypeStruct((N_EDGES,), jnp.float32),
            jax.ShapeDtypeStruct((N_EDGES,), jnp.float32),
            jax.ShapeDtypeStruct((NW, S0P), jnp.float32),
            jax.ShapeDtypeStruct((NW, G), jnp.float32),
            jax.ShapeDtypeStruct((NW, S0P), jnp.float32),
            jax.ShapeDtypeStruct((NW, G), jnp.float32),
            jax.ShapeDtypeStruct((NW, L), jnp.float32),
            jax.ShapeDtypeStruct((NW, L), jnp.float32),
        ),
        mesh=_sc_mesh(),
        compiler_params=_SC_PARAMS,
        scratch_types=[
            pltpu.VMEM((CH,), jnp.int32),
            pltpu.VMEM((CH,), jnp.float32),
            pltpu.VMEM((CH,), jnp.float32),
            pltpu.VMEM((CH,), jnp.float32),
            pltpu.VMEM((CH,), jnp.float32),
            pltpu.VMEM((N_NODES,), jnp.int32),
            pltpu.VMEM((S0P,), jnp.float32),
            pltpu.VMEM((G,), jnp.float32),
            pltpu.VMEM((CH,), jnp.float32),
            pltpu.VMEM((CH,), jnp.float32),
            pltpu.VMEM((S0P,), jnp.float32),
            pltpu.VMEM((G,), jnp.float32),
            pltpu.VMEM((S0P,), jnp.float32),
            pltpu.VMEM((G,), jnp.float32),
            pltpu.VMEM((L,), jnp.int32),
            pltpu.VMEM((L,), jnp.float32),
        ],
    )
    def k(idx_h, l0_h, l1_h, g0_h, g1_h, bv_h, m0f_h, m2f_h, tf_h,
          s0_h, s1_h, d0p_h, d2p_h, g0p_h, g2p_h, dp0_h, dp1_h,
          idx_v, l0_v, l1_v, g0_v, g1_v, bv_v, m0f_v, m2f_v, s0_v, s1_v,
          d0a, d2a, g0a, g2a, tf_v, dt_v):
        wid = lax.axis_index("s") * 2 + lax.axis_index("c")
        base = wid * CH
        pltpu.sync_copy(idx_h.at[pl.ds(base, CH)], idx_v)
        pltpu.sync_copy(l0_h.at[pl.ds(base, CH)], l0_v)
        pltpu.sync_copy(l1_h.at[pl.ds(base, CH)], l1_v)
        pltpu.sync_copy(g0_h.at[pl.ds(base, CH)], g0_v)
        pltpu.sync_copy(g1_h.at[pl.ds(base, CH)], g1_v)
        pltpu.sync_copy(bv_h, bv_v)
        pltpu.sync_copy(m0f_h, m0f_v)
        pltpu.sync_copy(m2f_h, m2f_v)
        pltpu.sync_copy(tf_h, tf_v)
        _vfill(d0a, S0P, 0.0, jnp.float32)
        _vfill(d2a, G, 0.0, jnp.float32)
        _vfill(g0a, S0P, 0.0, jnp.float32)
        _vfill(g2a, G, 0.0, jnp.float32)
        tfv = tf_v[pl.ds(0, L)] != 0

        def step(j, carry):
            dv0, dv1 = carry
            sl = pl.ds(j * L, L)
            idx = idx_v[sl]
            v0 = l0_v[sl]
            gv0 = g0_v[sl]
            m0g = plsc.load_gather(m0f_v, [idx])
            s0_v[sl] = jnp.where(tfv, gv0, (v0 >= m0g).astype(jnp.float32))
            plsc.addupdate_scatter(d0a, [idx], jnp.exp(v0 - m0g))
            plsc.addupdate_scatter(g0a, [idx], gv0)
            idx2 = plsc.load_gather(bv_v, [idx])
            v1 = l1_v[sl]
            gv1 = g1_v[sl]
            m2g = plsc.load_gather(m2f_v, [idx2])
            s1_v[sl] = jnp.where(tfv, gv1, (v1 >= m2g).astype(jnp.float32))
            plsc.addupdate_scatter(d2a, [idx2], jnp.exp(v1 - m2g))
            plsc.addupdate_scatter(g2a, [idx2], gv1)
            return (dv0 + gv0 * v0, dv1 + gv1 * v1)

        zero = jnp.zeros((L,), jnp.float32)
        dv0, dv1 = lax.fori_loop(0, CH // L, step, (zero, zero))
        pltpu.sync_copy(s0_v, s0_h.at[pl.ds(base, CH)])
        pltpu.sync_copy(s1_v, s1_h.at[pl.ds(base, CH)])
        pltpu.sync_copy(d0a, d0p_h.at[wid])
        pltpu.sync_copy(d2a, d2p_h.at[wid])
        pltpu.sync_copy(g0a, g0p_h.at[wid])
        pltpu.sync_copy(g2a, g2p_h.at[wid])
        dt_v[pl.ds(0, L)] = dv0
        pltpu.sync_copy(dt_v, dp0_h.at[wid])
        dt_v[pl.ds(0, L)] = dv1
        pltpu.sync_copy(dt_v, dp1_h.at[wid])

    return k(e_idx, l0, l1, g0, g1, batch_vec, m0f, m2f, tf16)


def _c2_body(d0p_ref, d2p_ref, m0f_ref, g0p_ref, m2f_ref, g2p_ref,
             dp0_ref, dp1_ref, np_ref, out_ref):
    d0f = jnp.sum(d0p_ref[...], axis=0, keepdims=True)
    d2f = jnp.sum(d2p_ref[...], axis=0, keepdims=True)
    g0f = jnp.sum(g0p_ref[...], axis=0, keepdims=True)
    g2f = jnp.sum(g2p_ref[...], axis=0, keepdims=True)
    dot0 = jnp.sum(dp0_ref[...])
    dot1 = jnp.sum(dp1_ref[...])
    npv = np_ref[...]
    ii = lax.broadcasted_iota(jnp.int32, (1, G), 1)
    loss_node = jnp.sum(jnp.where(ii == 0, npv, 0.0))
    n0count = jnp.sum(jnp.where(ii == 1, npv, 0.0))
    m0f = m0f_ref[...]
    m2f = m2f_ref[...]
    loss_a = (-dot0 + jnp.sum(g0f * m0f) +
              jnp.sum(g0f * jnp.log(d0f + 1e-20))) / N_NODES
    loss_b = n0count * (-dot1 + jnp.sum(g2f * m2f) +
                        jnp.sum(g2f * jnp.log(d2f + 1e-20))) / G
    out_ref[...] = jnp.full((1, 1), loss_node + loss_a + loss_b, jnp.float32)


def _c2(d0p, d2p, m0f, g0p, m2f, g2p, dp0, dp1, npart):
    return pl.pallas_call(
        _c2_body,
        grid=(1,),
        in_specs=[
            pl.BlockSpec((NW, S0P), lambda i: (0, 0)),
            pl.BlockSpec((NW, G), lambda i: (0, 0)),
            pl.BlockSpec((1, S0P), lambda i: (0, 0)),
            pl.BlockSpec((NW, S0P), lambda i: (0, 0)),
            pl.BlockSpec((1, G), lambda i: (0, 0)),
            pl.BlockSpec((NW, G), lambda i: (0, 0)),
            pl.BlockSpec((NW, L), lambda i: (0, 0)),
            pl.BlockSpec((NW, L), lambda i: (0, 0)),
            pl.BlockSpec((1, G), lambda i: (0, 0)),
        ],
        out_specs=pl.BlockSpec((1, 1), lambda i: (0, 0)),
        out_shape=jax.ShapeDtypeStruct((1, 1), jnp.float32),
    )(d0p, d2p, m0f, g0p, m2f, g2p, dp0, dp1, npart)


def kernel(node_fts, edge_fts, node_hints, edge_hints, W_node, b_node, W_edge,
           b_edge, batch_vec, edge_index, processor_step, training_step,
           teacher_force):
    step = jnp.asarray(processor_step, jnp.int32)
    tf_i = jnp.asarray(teacher_force, jnp.int32).reshape(1)
    tf16 = jnp.broadcast_to(tf_i, (L,))
    batch_vec = batch_vec.astype(jnp.int32)

    g0 = lax.dynamic_slice(edge_hints, (0, step, 0),
                           (N_EDGES, 1, 1)).reshape(N_EDGES)
    g1 = lax.dynamic_slice(edge_hints, (0, step, 1),
                           (N_EDGES, 1, 1)).reshape(N_EDGES)
    gtn = [lax.dynamic_slice(node_hints, (0, step, k),
                             (N_NODES, 1, 1)).reshape(N_NODES, 1)
           for k in range(3)]

    l0, l1, e_idx = _edge_logits(edge_fts, W_edge, b_edge,
                                 edge_index.astype(jnp.int32))

    m0p, m1p = _k1_partials(e_idx, l0, l1)
    states_n, npart = _c1a(node_fts, W_node, b_node, gtn[0], gtn[1], gtn[2],
                           batch_vec, tf_i)
    bv_row = jnp.pad(batch_vec, (0, S0P - N_NODES)).reshape(1, S0P)
    m0f, m2f = _c1b(m0p, m1p, bv_row)
    s0, s1, d0p, d2p, g0p, g2p, dp0, dp1 = _k3_denoms_states(
        e_idx, l0, l1, g0, g1, batch_vec, m0f.reshape(S0P), m2f.reshape(G),
        tf16)
    loss11 = _c2(d0p, d2p, m0f, g0p, m2f.reshape(1, G), g2p, dp0, dp1, npart)

    loss = loss11[0, 0]
    states_e = jnp.stack([s0, s1], axis=-1)
    return (states_n, states_e, loss)


# speedup vs baseline: 1.0440x; 1.0440x over previous
import functools

import jax
import jax.numpy as jnp
from jax import lax
from jax.experimental import pallas as pl
from jax.experimental.pallas import tpu as pltpu
from jax.experimental.pallas import tpu_sc as plsc

N_NODES = 10000
N_EDGES = 320000
H = 128
G = 128
EBLK = 16384
S0P = 10112
NW = 32
CH = N_EDGES // NW
L = 16
NEG = -3.4e38

_SC_PARAMS = pltpu.CompilerParams(needs_layout_passes=False)


def _sc_mesh():
    return plsc.VectorSubcoreMesh(
        core_axis_name="c", subcore_axis_name="s", num_cores=2, num_subcores=16)


def _a_body(fts_ref, w_ref, b_ref, ei_ref, l0_ref, l1_ref, idx_ref):
    lg = lax.dot_general(w_ref[...], fts_ref[...],
                         (((1,), (1,)), ((), ())))
    lg = lg + b_ref[...]
    l0_ref[...] = lg[0]
    l1_ref[...] = lg[1]
    idx_ref[...] = ei_ref[0]


NE_A = 163840
NE_B = N_EDGES - NE_A
BOFF = NE_A // EBLK


def _edge_logits(edge_fts, W_edge, b_edge, edge_index, n_out, boff):
    return pl.pallas_call(
        _a_body,
        grid=((n_out + EBLK - 1) // EBLK,),
        in_specs=[
            pl.BlockSpec((EBLK, H), lambda i: (i + boff, 0)),
            pl.BlockSpec((2, H), lambda i: (0, 0)),
            pl.BlockSpec((2, 1), lambda i: (0, 0)),
            pl.BlockSpec((2, EBLK), lambda i: (0, i + boff)),
        ],
        out_specs=[
            pl.BlockSpec((EBLK,), lambda i: (i,)),
            pl.BlockSpec((EBLK,), lambda i: (i,)),
            pl.BlockSpec((EBLK,), lambda i: (i,)),
        ],
        out_shape=[
            jax.ShapeDtypeStruct((n_out,), jnp.float32),
            jax.ShapeDtypeStruct((n_out,), jnp.float32),
            jax.ShapeDtypeStruct((n_out,), jnp.int32),
        ],
    )(edge_fts, W_edge, b_edge.reshape(2, 1), edge_index)


def _scatter_max16(acc, idx, val):
    def cond(act):
        return jnp.any(act)

    def body(act):
        cur = plsc.load_gather(acc, [idx])
        need = jnp.logical_and(act, val > cur)
        plsc.store_scatter(acc, [idx], val, mask=need)
        cur2 = plsc.load_gather(acc, [idx])
        return jnp.logical_and(need, val > cur2)

    act0 = val > plsc.load_gather(acc, [idx])
    lax.while_loop(cond, body, act0)


def _scatter_max16_pair(acc_a, idx_a, val_a, acc_b, idx_b, val_b):
    def cond(st):
        aa, ab = st
        return jnp.any(jnp.logical_or(aa, ab))

    def body(st):
        aa, ab = st
        cura = plsc.load_gather(acc_a, [idx_a])
        needa = jnp.logical_and(aa, val_a > cura)
        plsc.store_scatter(acc_a, [idx_a], val_a, mask=needa)
        curb = plsc.load_gather(acc_b, [idx_b])
        needb = jnp.logical_and(ab, val_b > curb)
        plsc.store_scatter(acc_b, [idx_b], val_b, mask=needb)
        cura2 = plsc.load_gather(acc_a, [idx_a])
        curb2 = plsc.load_gather(acc_b, [idx_b])
        return (jnp.logical_and(needa, val_a > cura2),
                jnp.logical_and(needb, val_b > curb2))

    aa0 = val_a > plsc.load_gather(acc_a, [idx_a])
    ab0 = val_b > plsc.load_gather(acc_b, [idx_b])
    lax.while_loop(cond, body, (aa0, ab0))


def _vfill(ref, n, value, dtype):
    def body(i, _):
        ref[pl.ds(i * L, L)] = jnp.full((L,), value, dtype)
        return 0

    lax.fori_loop(0, n // L, body, 0)


def _k1_partials(e_idx, l0, l1, ch):
    @functools.partial(
        pl.kernel,
        out_type=(
            jax.ShapeDtypeStruct((NW, S0P), jnp.float32),
            jax.ShapeDtypeStruct((NW, S0P), jnp.float32),
        ),
        mesh=_sc_mesh(),
        compiler_params=_SC_PARAMS,
        scratch_types=[
            pltpu.VMEM((ch,), jnp.int32),
            pltpu.VMEM((ch,), jnp.float32),
            pltpu.VMEM((ch,), jnp.float32),
            pltpu.VMEM((S0P,), jnp.float32),
            pltpu.VMEM((S0P,), jnp.float32),
        ],
    )
    def k(idx_h, l0_h, l1_h, m0p_h, m1p_h,
          idx_v, l0_v, l1_v, m0a, m1a):
        wid = lax.axis_index("s") * 2 + lax.axis_index("c")
        base = wid * ch
        pltpu.sync_copy(idx_h.at[pl.ds(base, ch)], idx_v)
        pltpu.sync_copy(l0_h.at[pl.ds(base, ch)], l0_v)
        pltpu.sync_copy(l1_h.at[pl.ds(base, ch)], l1_v)
        _vfill(m0a, S0P, NEG, jnp.float32)
        _vfill(m1a, S0P, NEG, jnp.float32)

        def step(j, _):
            sl = pl.ds(j * L, L)
            idx = idx_v[sl]
            _scatter_max16_pair(m0a, idx, l0_v[sl], m1a, idx, l1_v[sl])
            return 0

        lax.fori_loop(0, ch // L, step, 0)
        pltpu.sync_copy(m0a, m0p_h.at[wid])
        pltpu.sync_copy(m1a, m1p_h.at[wid])

    return k(e_idx, l0, l1)


_CB = 2500


def _c1a_chunk(c, nf_ref, w_ref, b_ref, g0_ref, g1_ref, g2_ref, bv_ref):
    sl = pl.ds(c * _CB, _CB)
    x = nf_ref[sl, :]
    logits = lax.dot_general(x, w_ref[...],
                             (((1,), (1,)), ((), ())))
    logits = logits + b_ref[...]
    gt = jnp.concatenate([g0_ref[sl, :], g1_ref[sl, :], g2_ref[sl, :]],
                         axis=1)
    bv = bv_ref[sl, :]
    onehot = bv == lax.broadcasted_iota(jnp.int32, (_CB, G), 1)
    return logits, gt, onehot


def _c1a_body(tf_ref, nf_ref, w_ref, b_ref, g0_ref, g1_ref, g2_ref, bv_ref,
              states_ref, np_ref):
    def ph1(c, carry):
        m_n, gseg, dotn, bce1, bce2, n0c = carry
        logits, gt, onehot = _c1a_chunk(c, nf_ref, w_ref, b_ref, g0_ref,
                                        g1_ref, g2_ref, bv_ref)
        l0 = logits[:, 0:1]
        g0 = gt[:, 0:1]
        m_n = jnp.maximum(m_n, jnp.max(jnp.where(onehot, l0, NEG), axis=0,
                                       keepdims=True))
        gseg = gseg + jnp.sum(jnp.where(onehot, g0, 0.0), axis=0,
                              keepdims=True)
        dotn = dotn + jnp.sum(g0 * l0)
        l1 = logits[:, 1:2]
        g1 = gt[:, 1:2]
        bce1 = bce1 + jnp.sum(jnp.maximum(l1, 0.0) - l1 * g1 +
                              jnp.log1p(jnp.exp(-jnp.abs(l1))))
        l2 = logits[:, 2:3]
        g2 = gt[:, 2:3]
        bce2 = bce2 + jnp.sum(jnp.maximum(l2, 0.0) - l2 * g2 +
                              jnp.log1p(jnp.exp(-jnp.abs(l2))))
        n0c = n0c + jnp.sum(jnp.where(onehot[:, 0:1], 1.0, 0.0))
        return m_n, gseg, dotn, bce1, bce2, n0c

    init = (jnp.full((1, G), NEG, jnp.float32),
            jnp.zeros((1, G), jnp.float32),
            jnp.float32(0.0), jnp.float32(0.0), jnp.float32(0.0),
            jnp.float32(0.0))
    m_n, gseg, dotn, bce1, bce2, n0c = lax.fori_loop(
        0, N_NODES // _CB, ph1, init)

    def ph2(c, denom):
        logits, gt, onehot = _c1a_chunk(c, nf_ref, w_ref, b_ref, g0_ref,
                                        g1_ref, g2_ref, bv_ref)
        l0 = logits[:, 0:1]
        m_gath = jnp.sum(jnp.where(onehot, m_n, 0.0), axis=1, keepdims=True)
        denom = denom + jnp.sum(jnp.where(onehot, jnp.exp(l0 - m_gath), 0.0),
                                axis=0, keepdims=True)
        preds = jnp.concatenate(
            [(l0 >= m_gath).astype(jnp.float32),
             (logits[:, 1:2] > 0.0).astype(jnp.float32),
             (logits[:, 2:3] > 0.0).astype(jnp.float32)], axis=1)
        states_ref[pl.ds(c * _CB, _CB), :] = jnp.where(tf_ref[0] != 0, gt,
                                                       preds)
        return denom

    denom = lax.fori_loop(0, N_NODES // _CB, ph2,
                          jnp.zeros((1, G), jnp.float32))

    loss_n0 = (-dotn + jnp.sum(gseg * m_n) +
               jnp.sum(gseg * jnp.log(denom + 1e-20))) / G
    loss_node = loss_n0 + bce1 / N_NODES + bce2 / N_NODES
    ii = lax.broadcasted_iota(jnp.int32, (1, G), 1)
    np_ref[...] = jnp.where(ii == 0, loss_node,
                            jnp.where(ii == 1, n0c, 0.0))


def _c1a(node_fts, W_node, b_node, gtn0, gtn1, gtn2, batch_vec, tf_i):
    return pl.pallas_call(
        _c1a_body,
        grid=(1,),
        in_specs=[
            pl.BlockSpec(memory_space=pltpu.SMEM),
            pl.BlockSpec((N_NODES, H), lambda i: (0, 0)),
            pl.BlockSpec((3, H), lambda i: (0, 0)),
            pl.BlockSpec((1, 3), lambda i: (0, 0)),
            pl.BlockSpec((N_NODES, 1), lambda i: (0, 0)),
            pl.BlockSpec((N_NODES, 1), lambda i: (0, 0)),
            pl.BlockSpec((N_NODES, 1), lambda i: (0, 0)),
            pl.BlockSpec((N_NODES, 1), lambda i: (0, 0)),
        ],
        out_specs=[
            pl.BlockSpec((N_NODES, 3), lambda i: (0, 0)),
            pl.BlockSpec((1, G), lambda i: (0, 0)),
        ],
        out_shape=[
            jax.ShapeDtypeStruct((N_NODES, 3), jnp.float32),
            jax.ShapeDtypeStruct((1, G), jnp.float32),
        ],
    )(tf_i, node_fts, W_node, b_node.reshape(1, 3), gtn0, gtn1, gtn2,
      batch_vec.reshape(N_NODES, 1))


def _c1b_body(m0pa_ref, m0pb_ref, m1pa_ref, m1pb_ref, bv_ref,
              m0f_ref, m2f_ref):
    m0f_ref[...] = jnp.maximum(
        jnp.max(m0pa_ref[...], axis=0, keepdims=True),
        jnp.max(m0pb_ref[...], axis=0, keepdims=True))
    m1f = jnp.maximum(
        jnp.max(m1pa_ref[...], axis=0, keepdims=True),
        jnp.max(m1pb_ref[...], axis=0, keepdims=True))
    gmask = bv_ref[...] == lax.broadcasted_iota(jnp.int32, (G, S0P), 0)
    m2f_ref[...] = jnp.max(jnp.where(gmask, m1f, NEG), axis=1, keepdims=True)


def _c1b(m0pa, m0pb, m1pa, m1pb, bv_row):
    return pl.pallas_call(
        _c1b_body,
        grid=(1,),
        in_specs=[
            pl.BlockSpec((NW, S0P), lambda i: (0, 0)),
            pl.BlockSpec((NW, S0P), lambda i: (0, 0)),
            pl.BlockSpec((NW, S0P), lambda i: (0, 0)),
            pl.BlockSpec((NW, S0P), lambda i: (0, 0)),
            pl.BlockSpec((1, S0P), lambda i: (0, 0)),
        ],
        out_specs=[
            pl.BlockSpec((1, S0P), lambda i: (0, 0)),
            pl.BlockSpec((G, 1), lambda i: (0, 0)),
        ],
        out_shape=[
            jax.ShapeDtypeStruct((1, S0P), jnp.float32),
            jax.ShapeDtypeStruct((G, 1), jnp.float32),
        ],
    )(m0pa, m0pb, m1pa, m1pb, bv_row)


def _k3_denoms_states(e_idx, l0, l1, g0, g1, batch_vec, m0f, m2f, tf16):
    @functools.partial(
        pl.kernel,
        out_type=(
            jax.ShapeDtypeStruct((N_EDGES,), jnp.float32),
            jax.ShapeDtypeStruct((N_EDGES,), jnp.float32),
            jax.ShapeDtypeStruct((NW, S0P), jnp.float32),
            jax.ShapeDtypeStruct((NW, G), jnp.float32),
            jax.ShapeDtypeStruct((NW, S0P), jnp.float32),
            jax.ShapeDtypeStruct((NW, G), jnp.float32),
            jax.ShapeDtypeStruct((NW, L), jnp.float32),
            jax.ShapeDtypeStruct((NW, L), jnp.float32),
        ),
        mesh=_sc_mesh(),
        compiler_params=_SC_PARAMS,
        scratch_types=[
            pltpu.VMEM((CH,), jnp.int32),
            pltpu.VMEM((CH,), jnp.float32),
            pltpu.VMEM((CH,), jnp.float32),
            pltpu.VMEM((CH,), jnp.float32),
            pltpu.VMEM((CH,), jnp.float32),
            pltpu.VMEM((N_NODES,), jnp.int32),
            pltpu.VMEM((S0P,), jnp.float32),
            pltpu.VMEM((G,), jnp.float32),
            pltpu.VMEM((CH,), jnp.float32),
            pltpu.VMEM((CH,), jnp.float32),
            pltpu.VMEM((S0P,), jnp.float32),
            pltpu.VMEM((G,), jnp.float32),
            pltpu.VMEM((S0P,), jnp.float32),
            pltpu.VMEM((G,), jnp.float32),
            pltpu.VMEM((L,), jnp.int32),
            pltpu.VMEM((L,), jnp.float32),
        ],
    )
    def k(idx_h, l0_h, l1_h, g0_h, g1_h, bv_h, m0f_h, m2f_h, tf_h,
          s0_h, s1_h, d0p_h, d2p_h, g0p_h, g2p_h, dp0_h, dp1_h,
          idx_v, l0_v, l1_v, g0_v, g1_v, bv_v, m0f_v, m2f_v, s0_v, s1_v,
          d0a, d2a, g0a, g2a, tf_v, dt_v):
        wid = lax.axis_index("s") * 2 + lax.axis_index("c")
        base = wid * CH
        pltpu.sync_copy(idx_h.at[pl.ds(base, CH)], idx_v)
        pltpu.sync_copy(l0_h.at[pl.ds(base, CH)], l0_v)
        pltpu.sync_copy(l1_h.at[pl.ds(base, CH)], l1_v)
        pltpu.sync_copy(g0_h.at[pl.ds(base, CH)], g0_v)
        pltpu.sync_copy(g1_h.at[pl.ds(base, CH)], g1_v)
        pltpu.sync_copy(bv_h, bv_v)
        pltpu.sync_copy(m0f_h, m0f_v)
        pltpu.sync_copy(m2f_h, m2f_v)
        pltpu.sync_copy(tf_h, tf_v)
        _vfill(d0a, S0P, 0.0, jnp.float32)
        _vfill(d2a, G, 0.0, jnp.float32)
        _vfill(g0a, S0P, 0.0, jnp.float32)
        _vfill(g2a, G, 0.0, jnp.float32)
        tfv = tf_v[pl.ds(0, L)] != 0

        def step(j, carry):
            dv0, dv1 = carry
            sl = pl.ds(j * L, L)
            idx = idx_v[sl]
            v0 = l0_v[sl]
            gv0 = g0_v[sl]
            m0g = plsc.load_gather(m0f_v, [idx])
            s0_v[sl] = jnp.where(tfv, gv0, (v0 >= m0g).astype(jnp.float32))
            plsc.addupdate_scatter(d0a, [idx], jnp.exp(v0 - m0g))
            plsc.addupdate_scatter(g0a, [idx], gv0)
            idx2 = plsc.load_gather(bv_v, [idx])
            v1 = l1_v[sl]
            gv1 = g1_v[sl]
            m2g = plsc.load_gather(m2f_v, [idx2])
            s1_v[sl] = jnp.where(tfv, gv1, (v1 >= m2g).astype(jnp.float32))
            plsc.addupdate_scatter(d2a, [idx2], jnp.exp(v1 - m2g))
            plsc.addupdate_scatter(g2a, [idx2], gv1)
            return (dv0 + gv0 * v0, dv1 + gv1 * v1)

        zero = jnp.zeros((L,), jnp.float32)
        dv0, dv1 = lax.fori_loop(0, CH // L, step, (zero, zero))
        pltpu.sync_copy(s0_v, s0_h.at[pl.ds(base, CH)])
        pltpu.sync_copy(s1_v, s1_h.at[pl.ds(base, CH)])
        pltpu.sync_copy(d0a, d0p_h.at[wid])
        pltpu.sync_copy(d2a, d2p_h.at[wid])
        pltpu.sync_copy(g0a, g0p_h.at[wid])
        pltpu.sync_copy(g2a, g2p_h.at[wid])
        dt_v[pl.ds(0, L)] = dv0
        pltpu.sync_copy(dt_v, dp0_h.at[wid])
        dt_v[pl.ds(0, L)] = dv1
        pltpu.sync_copy(dt_v, dp1_h.at[wid])

    return k(e_idx, l0, l1, g0, g1, batch_vec, m0f, m2f, tf16)


def _c2_body(d0p_ref, d2p_ref, m0f_ref, g0p_ref, m2f_ref, g2p_ref,
             dp0_ref, dp1_ref, np_ref, out_ref):
    d0f = jnp.sum(d0p_ref[...], axis=0, keepdims=True)
    d2f = jnp.sum(d2p_ref[...], axis=0, keepdims=True)
    g0f = jnp.sum(g0p_ref[...], axis=0, keepdims=True)
    g2f = jnp.sum(g2p_ref[...], axis=0, keepdims=True)
    dot0 = jnp.sum(dp0_ref[...])
    dot1 = jnp.sum(dp1_ref[...])
    npv = np_ref[...]
    ii = lax.broadcasted_iota(jnp.int32, (1, G), 1)
    loss_node = jnp.sum(jnp.where(ii == 0, npv, 0.0))
    n0count = jnp.sum(jnp.where(ii == 1, npv, 0.0))
    m0f = m0f_ref[...]
    m2f = m2f_ref[...]
    loss_a = (-dot0 + jnp.sum(g0f * m0f) +
              jnp.sum(g0f * jnp.log(d0f + 1e-20))) / N_NODES
    loss_b = n0count * (-dot1 + jnp.sum(g2f * m2f) +
                        jnp.sum(g2f * jnp.log(d2f + 1e-20))) / G
    out_ref[...] = jnp.full((1, 1), loss_node + loss_a + loss_b, jnp.float32)


def _c2(d0p, d2p, m0f, g0p, m2f, g2p, dp0, dp1, npart):
    return pl.pallas_call(
        _c2_body,
        grid=(1,),
        in_specs=[
            pl.BlockSpec((NW, S0P), lambda i: (0, 0)),
            pl.BlockSpec((NW, G), lambda i: (0, 0)),
            pl.BlockSpec((1, S0P), lambda i: (0, 0)),
            pl.BlockSpec((NW, S0P), lambda i: (0, 0)),
            pl.BlockSpec((1, G), lambda i: (0, 0)),
            pl.BlockSpec((NW, G), lambda i: (0, 0)),
            pl.BlockSpec((NW, L), lambda i: (0, 0)),
            pl.BlockSpec((NW, L), lambda i: (0, 0)),
            pl.BlockSpec((1, G), lambda i: (0, 0)),
        ],
        out_specs=pl.BlockSpec((1, 1), lambda i: (0, 0)),
        out_shape=jax.ShapeDtypeStruct((1, 1), jnp.float32),
    )(d0p, d2p, m0f, g0p, m2f, g2p, dp0, dp1, npart)


def kernel(node_fts, edge_fts, node_hints, edge_hints, W_node, b_node, W_edge,
           b_edge, batch_vec, edge_index, processor_step, training_step,
           teacher_force):
    step = jnp.asarray(processor_step, jnp.int32)
    tf_i = jnp.asarray(teacher_force, jnp.int32).reshape(1)
    tf16 = jnp.broadcast_to(tf_i, (L,))
    batch_vec = batch_vec.astype(jnp.int32)

    g0 = lax.dynamic_slice(edge_hints, (0, step, 0),
                           (N_EDGES, 1, 1)).reshape(N_EDGES)
    g1 = lax.dynamic_slice(edge_hints, (0, step, 1),
                           (N_EDGES, 1, 1)).reshape(N_EDGES)
    gtn = [lax.dynamic_slice(node_hints, (0, step, k),
                             (N_NODES, 1, 1)).reshape(N_NODES, 1)
           for k in range(3)]

    ei32 = edge_index.astype(jnp.int32)
    l0a, l1a, idxa = _edge_logits(edge_fts, W_edge, b_edge, ei32, NE_A, 0)
    m0pa, m1pa = _k1_partials(idxa, l0a, l1a, NE_A // NW)
    l0b, l1b, idxb = _edge_logits(edge_fts, W_edge, b_edge, ei32, NE_B, BOFF)
    m0pb, m1pb = _k1_partials(idxb, l0b, l1b, NE_B // NW)
    l0 = jnp.concatenate([l0a, l0b])
    l1 = jnp.concatenate([l1a, l1b])
    e_idx = jnp.concatenate([idxa, idxb])
    states_n, npart = _c1a(node_fts, W_node, b_node, gtn[0], gtn[1], gtn[2],
                           batch_vec, tf_i)
    bv_row = jnp.pad(batch_vec, (0, S0P - N_NODES)).reshape(1, S0P)
    m0f, m2f = _c1b(m0pa, m0pb, m1pa, m1pb, bv_row)
    s0, s1, d0p, d2p, g0p, g2p, dp0, dp1 = _k3_denoms_states(
        e_idx, l0, l1, g0, g1, batch_vec, m0f.reshape(S0P), m2f.reshape(G),
        tf16)
    loss11 = _c2(d0p, d2p, m0f, g0p, m2f.reshape(1, G), g2p, dp0, dp1, npart)

    loss = loss11[0, 0]
    states_e = jnp.stack([s0, s1], axis=-1)
    return (states_n, states_e, loss)
